# Initial kernel scaffold; baseline (speedup 1.0000x reference)
import jax
import jax.numpy as jnp
from jax.experimental import pallas as pl


def kernel(x, indices):
    raise NotImplementedError("write your pallas kernel here")


# speedup vs baseline: 1.6619x; 1.6619x over previous
import jax
import jax.numpy as jnp
from jax.experimental import pallas as pl
from jax.experimental.pallas import tpu as pltpu

_KH, _KW = 2, 2


def _body(sidx, rr, kk, ss, x_ref, o_ref):
    t = pl.program_id(0)
    idx = sidx[t]
    di = (idx // _KW) % _KH
    dj = idx % _KW
    oh = o_ref.shape[-2]
    ow = o_ref.shape[-1]
    for djv in range(_KW):

        @pl.when(dj == djv)
        def _(djv=djv):
            o_ref[0, :, 0, 0, :, :] = x_ref[:, 0, pl.ds(di, oh), djv : djv + ow]


def kernel(x, indices):
    B, C, H, W = x.shape
    lut_rank, num_kernels, sample_size = indices.shape
    oh, ow = H - _KH + 1, W - _KW + 1
    idxf = indices.reshape(-1).astype(jnp.int32)
    chan = idxf // (_KH * _KW)
    order = jnp.argsort(chan).astype(jnp.int32)
    sidx = idxf[order]
    rr = order // (num_kernels * sample_size)
    kk = (order // sample_size) % num_kernels
    ss = order % sample_size
    nslots = idxf.shape[0]

    grid_spec = pltpu.PrefetchScalarGridSpec(
        num_scalar_prefetch=4,
        grid=(nslots,),
        in_specs=[
            pl.BlockSpec(
                (B, 1, H, W),
                lambda t, sidx, rr, kk, ss: (0, sidx[t] // (_KH * _KW), 0, 0),
            )
        ],
        out_specs=pl.BlockSpec(
            (1, B, 1, 1, oh, ow),
            lambda t, sidx, rr, kk, ss: (rr[t], 0, kk[t], ss[t], 0, 0),
        ),
    )
    out = pl.pallas_call(
        _body,
        grid_spec=grid_spec,
        out_shape=jax.ShapeDtypeStruct(
            (lut_rank, B, num_kernels, sample_size, oh, ow), x.dtype
        ),
    )(sidx, rr, kk, ss, x)
    return out.reshape(lut_rank, B, num_kernels, sample_size, oh * ow)
